# Initial kernel scaffold; baseline (speedup 1.0000x reference)
import jax
import jax.numpy as jnp
from jax.experimental import pallas as pl


def kernel(val_tok, embedding_weight):
    raise NotImplementedError("write your pallas kernel here")


# speedup vs baseline: 4.5460x; 4.5460x over previous
import functools

import jax
import jax.numpy as jnp
from jax import lax
from jax.experimental import pallas as pl
from jax.experimental.pallas import tpu as pltpu
from jax.experimental.pallas import tpu_sc as plsc

VOCAB = 100000
N_WORD = 64
B = 4096
L = 50

_INFO = plsc.get_sparse_core_info()
_NC = _INFO.num_cores
_NS = _INFO.num_subcores
_NW = _NC * _NS
_TOT = B * L
_PER_W = _TOT // _NW
_CHUNK = 800
_NCHUNK = _PER_W // _CHUNK


@functools.partial(
    pl.kernel,
    mesh=plsc.VectorSubcoreMesh(core_axis_name="c", subcore_axis_name="s"),
    out_type=jax.ShapeDtypeStruct((_TOT, N_WORD), jnp.float32),
    scratch_types=[
        pltpu.VMEM((_CHUNK,), jnp.int32),
        pltpu.VMEM((_CHUNK, N_WORD), jnp.float32),
        pltpu.SemaphoreType.DMA,
    ],
    compiler_params=pltpu.CompilerParams(use_tc_tiling_on_sc=False),
)
def _gather_kernel(idx_hbm, table_hbm, out_hbm, idx_v, rows_v, sem):
    wid = lax.axis_index("s") * _NC + lax.axis_index("c")
    base = wid * _PER_W
    for j in range(_NCHUNK):
        off = base + j * _CHUNK
        pltpu.sync_copy(idx_hbm.at[pl.ds(off, _CHUNK)], idx_v)
        pltpu.async_copy(table_hbm.at[idx_v], rows_v, sem).wait()
        pltpu.sync_copy(rows_v, out_hbm.at[pl.ds(off, _CHUNK)])


def kernel(val_tok, embedding_weight):
    flat_idx = val_tok.reshape(_TOT).astype(jnp.int32)
    out = _gather_kernel(flat_idx, embedding_weight)
    return out.reshape(B, L, N_WORD)


# speedup vs baseline: 4.6984x; 1.0335x over previous
import functools

import jax
import jax.numpy as jnp
from jax import lax
from jax.experimental import pallas as pl
from jax.experimental.pallas import tpu as pltpu
from jax.experimental.pallas import tpu_sc as plsc

VOCAB = 100000
N_WORD = 64
B = 4096
L = 50

_INFO = plsc.get_sparse_core_info()
_NC = _INFO.num_cores
_NS = _INFO.num_subcores
_NW = _NC * _NS
_TOT = B * L
_PER_W = _TOT // _NW
_CHUNK = 640
_NCHUNK = _PER_W // _CHUNK
_NBUF = 3


@functools.partial(
    pl.kernel,
    mesh=plsc.VectorSubcoreMesh(core_axis_name="c", subcore_axis_name="s"),
    out_type=jax.ShapeDtypeStruct((_TOT, N_WORD), jnp.float32),
    scratch_types=[
        pltpu.VMEM((_PER_W,), jnp.int32),
        [pltpu.VMEM((_CHUNK, N_WORD), jnp.float32) for _ in range(_NBUF)],
        [pltpu.SemaphoreType.DMA for _ in range(_NBUF)],
        [pltpu.SemaphoreType.DMA for _ in range(_NBUF)],
    ],
    compiler_params=pltpu.CompilerParams(use_tc_tiling_on_sc=False),
)
def _gather_kernel(idx_hbm, table_hbm, out_hbm, idx_v, rows, gsem, ssem):
    wid = lax.axis_index("s") * _NC + lax.axis_index("c")
    base = wid * _PER_W
    pltpu.sync_copy(idx_hbm.at[pl.ds(base, _PER_W)], idx_v)

    def fire_gather(j):
        b = j % _NBUF
        return pltpu.async_copy(
            table_hbm.at[idx_v.at[pl.ds(j * _CHUNK, _CHUNK)]], rows[b], gsem[b])

    def fire_store(j):
        b = j % _NBUF
        return pltpu.async_copy(
            rows[b], out_hbm.at[pl.ds(base + j * _CHUNK, _CHUNK)], ssem[b])

    gd = [None] * _NCHUNK
    sd = [None] * _NCHUNK
    for j in range(_NBUF):
        gd[j] = fire_gather(j)
    for j in range(_NCHUNK):
        nxt = j - 1 + _NBUF
        if j >= 1 and nxt < _NCHUNK:
            sd[j - 1].wait()
            gd[nxt] = fire_gather(nxt)
        gd[j].wait()
        sd[j] = fire_store(j)
    for j in range(max(0, _NCHUNK - _NBUF), _NCHUNK):
        sd[j].wait()


def kernel(val_tok, embedding_weight):
    flat_idx = val_tok.reshape(_TOT).astype(jnp.int32)
    out = _gather_kernel(flat_idx, embedding_weight)
    return out.reshape(B, L, N_WORD)
